# Initial kernel scaffold; baseline (speedup 1.0000x reference)
import jax
import jax.numpy as jnp
from jax.experimental import pallas as pl


def kernel(x, edge_index, edge_vals, W0, b):
    raise NotImplementedError("write your pallas kernel here")


# speedup vs baseline: 5.5747x; 5.5747x over previous
import functools

import jax
import jax.numpy as jnp
from jax import lax
from jax.experimental import pallas as pl
from jax.experimental.pallas import tpu as pltpu
from jax.experimental.pallas import tpu_sc as plsc

N = 10000
D = 256
E = 160000
K = 2000

NC = 2
NS = 16
L = 16

NPAD = 10240
CH = NPAD // NS
NV = CH // L
EPAD = 163840
EC = EPAD // (NC * NS)
ECH = EC // 128
KPAD = 2048

_I32 = jnp.int32
_F32 = jnp.float32


def _iota16():
  return lax.iota(_I32, 16)


def _ones16():
  return jnp.full((16,), 1, _I32)


def _matvec_body(x_ref, w_ref, o_ref):
  o_ref[...] = jax.lax.dot_general(
      x_ref[...], w_ref[...], (((1,), (0,)), ((), ())),
      preferred_element_type=_F32)


def _matvec(x, w0):
  return pl.pallas_call(
      _matvec_body,
      out_shape=jax.ShapeDtypeStruct((N, 1), _F32),
  )(x, w0)


_mesh2 = plsc.VectorSubcoreMesh(
    core_axis_name="c", subcore_axis_name="s", num_cores=NC)


_EDGE_KW = dict(
    out_type=jax.ShapeDtypeStruct((NC, NPAD), _F32),
    mesh=_mesh2,
    compiler_params=pltpu.CompilerParams(needs_layout_passes=False),
    scratch_types=[
        pltpu.VMEM_SHARED((NPAD,), _F32),
        pltpu.VMEM((N,), _F32),
        pltpu.VMEM((EC,), _I32),
        pltpu.VMEM((EC,), _F32),
        pltpu.VMEM((EC,), _F32),
        pltpu.VMEM((ECH, 128), _I32),
        pltpu.VMEM((CH,), _F32),
    ],
)


def _edge_body(ps_hbm, dst_hbm, ev_hbm, src2_hbm, part_hbm,
               sup_sp, ps_v, dst_v, ev_v, msg_v, src_v, z_v):
  c = lax.axis_index("c")
  s = lax.axis_index("s")
  w = c * NS + s
  base = w * EC

  pltpu.sync_copy(ps_hbm, ps_v)
  pltpu.sync_copy(dst_hbm.at[pl.ds(base, EC)], dst_v)
  pltpu.sync_copy(ev_hbm.at[pl.ds(base, EC)], ev_v)
  pltpu.sync_copy(src2_hbm.at[pl.ds(w * ECH, ECH)], src_v)

  def zero_body(i, carry):
    z_v[pl.ds(pl.multiple_of(i * 16, 16), 16)] = jnp.zeros((16,), _F32)
    return carry

  lax.fori_loop(0, CH // 16, zero_body, 0)
  pltpu.sync_copy(z_v, sup_sp.at[pl.ds(s * CH, CH)])
  plsc.subcore_barrier()

  def msg_body(i, carry):
    sl = pl.ds(pl.multiple_of(i * 16, 16), 16)
    d = dst_v[sl]
    g = plsc.load_gather(ps_v, [d])
    msg_v[sl] = g * ev_v[sl]
    return carry

  lax.fori_loop(0, EC // 16, msg_body, 0)

  def sc_body(j, carry):
    pltpu.sync_copy(msg_v.at[pl.ds(pl.multiple_of(j * 128, 128), 128)],
                    sup_sp.at[src_v.at[j]], add=True)
    return carry

  lax.fori_loop(0, ECH, sc_body, 0)
  plsc.subcore_barrier()

  pltpu.sync_copy(sup_sp.at[pl.ds(s * CH, CH)],
                  part_hbm.at[c, pl.ds(s * CH, CH)])


_edge_kernel = pl.kernel(_edge_body, **_EDGE_KW)


_mesh1 = plsc.VectorSubcoreMesh(
    core_axis_name="c", subcore_axis_name="s", num_cores=1)


@functools.partial(
    pl.kernel,
    out_type=(jax.ShapeDtypeStruct((KPAD,), _I32),
              jax.ShapeDtypeStruct((KPAD,), _F32)),
    mesh=_mesh1,
    compiler_params=pltpu.CompilerParams(needs_layout_passes=False),
    scratch_types=[
        pltpu.VMEM_SHARED((NPAD,), _I32),
        pltpu.VMEM_SHARED((NPAD,), _I32),
        pltpu.VMEM_SHARED((NPAD,), _I32),
        pltpu.VMEM_SHARED((NPAD,), _I32),
        pltpu.VMEM_SHARED((NS * 4096,), _I32),
        pltpu.VMEM_SHARED((NS * 4096,), _I32),
        pltpu.VMEM_SHARED((16, 16), _I32),
        pltpu.VMEM((CH,), _I32),
        pltpu.VMEM((CH,), _I32),
        pltpu.VMEM((CH,), _F32),
        pltpu.VMEM((CH,), _F32),
        pltpu.VMEM((4096,), _I32),
        pltpu.VMEM((4096,), _I32),
        pltpu.VMEM((4096,), _I32),
        pltpu.VMEM((32, 128), _I32),
        pltpu.VMEM((32, 128), _I32),
        pltpu.VMEM((5, 128), _I32),
        pltpu.VMEM((16,), _I32),
        pltpu.VMEM((16,), _I32),
        pltpu.VMEM((16, 16), _I32),
        pltpu.VMEM((16,), _F32),
        pltpu.VMEM((128,), _I32),
        pltpu.VMEM((128,), _F32),
    ],
)
def _sort_kernel(part_hbm, b_hbm, ids_hbm, vals_hbm,
                 keyA, idA, keyB, idB, histtab, offtab, totals,
                 ck_v, cid_v, p0_v, p1_v, hist_v, seg_v, off_v,
                 scanidx_v, myoffidx_v, posb_v, tmp16_v, t16_v, tot_v,
                 b_v, ek_v, evo_v):
  t = lax.axis_index("s")
  iota = _iota16()
  ones = _ones16()
  base_idx = iota * NV

  def _bcast(vec_i32, j):
    tmp16_v[...] = vec_i32
    return plsc.load_gather(
        tmp16_v, [jnp.broadcast_to(j, (16,)).astype(_I32)])

  def pre_body(i, carry):
    n = i * 16 + iota
    b = t * 16 + (n >> 8)
    tt = (n >> 4) & 15
    l = n & 15
    scanidx_v[i // 8, pl.ds(pl.multiple_of((i % 8) * 16, 16), 16)] = (
        tt * 4096 + b * 16 + l)
    d = n >> 4
    myoffidx_v[i // 8, pl.ds(pl.multiple_of((i % 8) * 16, 16), 16)] = (
        d * 256 + t * 16 + l)
    return carry

  lax.fori_loop(0, 256, pre_body, 0)

  pltpu.sync_copy(part_hbm.at[0, pl.ds(t * CH, CH)], p0_v)
  pltpu.sync_copy(part_hbm.at[1, pl.ds(t * CH, CH)], p1_v)
  pltpu.sync_copy(b_hbm, b_v)
  bb = b_v[...]

  def init_body(i, carry):
    sl = pl.ds(pl.multiple_of(i * 16, 16), 16)
    sc = p0_v[sl] + p1_v[sl] + bb
    u = lax.bitcast_convert_type(sc, _I32)
    key = jnp.where(u < 0, u, u ^ jnp.int32(0x7FFFFFFF))
    gidx = t * CH + i * 16 + iota
    key = jnp.where(gidx < N, key, jnp.int32(-1))
    ck_v[sl] = key
    cid_v[sl] = gidx
    return carry

  lax.fori_loop(0, NV, init_body, 0)
  pltpu.sync_copy(ck_v, keyA.at[pl.ds(t * CH, CH)])
  pltpu.sync_copy(cid_v, idA.at[pl.ds(t * CH, CH)])

  def radix_pass(src_key, src_id, dst_key, dst_id, shift):
    def hz_body(i, carry):
      hist_v[pl.ds(pl.multiple_of(i * 16, 16), 16)] = jnp.zeros((16,), _I32)
      return carry

    lax.fori_loop(0, 256, hz_body, 0)
    pltpu.sync_copy(src_key.at[pl.ds(t * CH, CH)], ck_v)
    pltpu.sync_copy(src_id.at[pl.ds(t * CH, CH)], cid_v)

    def hist_body(v, carry):
      kk = plsc.load_gather(ck_v, [base_idx + v])
      d = (kk >> shift) & 255
      plsc.addupdate_scatter(hist_v, [d * 16 + iota], ones)
      return carry

    lax.fori_loop(0, NV, hist_body, 0)
    pltpu.sync_copy(hist_v, histtab.at[pl.ds(t * 4096, 4096)])
    plsc.subcore_barrier()

    def sg_body(j, carry):
      pltpu.sync_copy(histtab.at[scanidx_v.at[j]],
                      seg_v.at[pl.ds(pl.multiple_of(j * 128, 128), 128)])
      return carry

    lax.fori_loop(0, 32, sg_body, 0)

    def scan_body(i, carry):
      sl = pl.ds(pl.multiple_of(i * 16, 16), 16)
      v = seg_v[sl]
      inc = plsc.cumsum(v)
      seg_v[sl] = inc - v + carry
      return carry + _bcast(inc, 15)

    my_total = lax.fori_loop(0, 256, scan_body, jnp.zeros((16,), _I32))
    t16_v[...] = my_total
    pltpu.sync_copy(t16_v, totals.at[t])
    plsc.subcore_barrier()
    pltpu.sync_copy(totals, tot_v)
    tot16 = plsc.load_gather(tot_v, [iota, jnp.zeros((16,), _I32)])
    exc = plsc.cumsum(tot16) - tot16
    myprefix = _bcast(exc, t)

    def addp_body(i, carry):
      sl = pl.ds(pl.multiple_of(i * 16, 16), 16)
      seg_v[sl] = seg_v[sl] + myprefix
      return carry

    lax.fori_loop(0, 256, addp_body, 0)

    def so_body(j, carry):
      pltpu.sync_copy(seg_v.at[pl.ds(pl.multiple_of(j * 128, 128), 128)],
                      offtab.at[scanidx_v.at[j]])
      return carry

    lax.fori_loop(0, 32, so_body, 0)
    plsc.subcore_barrier()

    def og_body(j, carry):
      pltpu.sync_copy(offtab.at[myoffidx_v.at[j]],
                      off_v.at[pl.ds(pl.multiple_of(j * 128, 128), 128)])
      return carry

    lax.fori_loop(0, 32, og_body, 0)

    def perm_body(v, carry):
      n = base_idx + v
      kk = plsc.load_gather(ck_v, [n])
      d = (kk >> shift) & 255
      a = d * 16 + iota
      pos = plsc.load_gather(off_v, [a])
      plsc.addupdate_scatter(off_v, [a], ones)
      plsc.store_scatter(posb_v, [n >> 7, n & 127], pos)
      return carry

    lax.fori_loop(0, NV, perm_body, 0)

    def pscat_body(j, carry):
      sl = pl.ds(pl.multiple_of(j * 128, 128), 128)
      pltpu.sync_copy(ck_v.at[sl], dst_key.at[posb_v.at[j]])
      pltpu.sync_copy(cid_v.at[sl], dst_id.at[posb_v.at[j]])
      return carry

    lax.fori_loop(0, CH // 128, pscat_body, 0)
    plsc.subcore_barrier()

  radix_pass(keyA, idA, keyB, idB, 0)
  radix_pass(keyB, idB, keyA, idA, 8)
  radix_pass(keyA, idA, keyB, idB, 16)
  radix_pass(keyB, idB, keyA, idA, 24)

  pltpu.sync_copy(keyA.at[pl.ds(t * 128, 128)], ek_v)

  def emit_body(i, carry):
    sl = pl.ds(pl.multiple_of(i * 16, 16), 16)
    key = ek_v[sl]
    u = jnp.where(key < 0, key, key ^ jnp.int32(0x7FFFFFFF))
    scv = lax.bitcast_convert_type(u, _F32)
    e = jnp.exp(scv * 2.0)
    evo_v[sl] = 1.0 - 2.0 / (e + 1.0)
    return carry

  lax.fori_loop(0, 8, emit_body, 0)
  pltpu.sync_copy(evo_v, vals_hbm.at[pl.ds(t * 128, 128)])
  pltpu.sync_copy(idA.at[pl.ds(t * 128, 128)],
                  ids_hbm.at[pl.ds(t * 128, 128)])


RW = KPAD // (NC * NS)


@functools.partial(
    pl.kernel,
    out_type=jax.ShapeDtypeStruct((KPAD * D,), _F32),
    mesh=_mesh2,
    compiler_params=pltpu.CompilerParams(
        needs_layout_passes=False, use_tc_tiling_on_sc=False),
    scratch_types=[
        pltpu.VMEM((RW,), _I32),
        pltpu.VMEM((RW,), _F32),
        pltpu.VMEM((RW * D,), _F32),
        pltpu.VMEM((16,), _F32),
        pltpu.SemaphoreType.DMA,
    ],
)
def _gather_kernel(xf_hbm, ids_hbm, vals_hbm, out_hbm,
                   id_v, val_v, rows_v, vtmp_v, sem):
  c = lax.axis_index("c")
  s = lax.axis_index("s")
  w = c * NS + s
  base = w * RW

  pltpu.sync_copy(ids_hbm.at[pl.ds(base, RW)], id_v)
  pltpu.sync_copy(vals_hbm.at[pl.ds(base, RW)], val_v)
  iota = _iota16()

  def grow_body(r, carry):
    g16 = id_v[pl.ds(pl.multiple_of((r >> 4) * 16, 16), 16)]
    rid = jnp.sum(jnp.where(iota == (r & 15), g16, 0))
    off = pl.multiple_of(rid * D, D)
    pltpu.sync_copy(xf_hbm.at[pl.ds(off, D)],
                    rows_v.at[pl.ds(pl.multiple_of(r * D, D), D)])
    return carry

  lax.fori_loop(0, RW, grow_body, 0)

  def row_body(r, carry):
    g16 = val_v[pl.ds(pl.multiple_of((r >> 4) * 16, 16), 16)]
    vtmp_v[...] = g16
    lane = jnp.broadcast_to(r & 15, (16,)).astype(_I32)
    vv = plsc.load_gather(vtmp_v, [lane])
    for cb in range(D // 16):
      sl = pl.ds(pl.multiple_of(r * D + cb * 16, 16), 16)
      rows_v[sl] = rows_v[sl] * vv
    return carry

  lax.fori_loop(0, RW, row_body, 0)
  pltpu.sync_copy(rows_v,
                  out_hbm.at[pl.ds(pl.multiple_of(base * D, D), RW * D)])


def kernel(x, edge_index, edge_vals, W0, b):
  src = edge_index[0]
  dst = edge_index[1]
  pad = EPAD - E
  dst_p = jnp.concatenate([dst, jnp.zeros((pad,), _I32)])
  src_p = jnp.concatenate([src, jnp.zeros((pad,), _I32)])
  ev_p = jnp.concatenate([edge_vals, jnp.zeros((pad,), _F32)])
  src2 = src_p.reshape(EPAD // 128, 128)
  w0t = W0.reshape(1, D)
  b16 = jnp.broadcast_to(b, (16,)).astype(_F32)

  pre = _matvec(x, W0)
  ps = pre.reshape(N)
  part = _edge_kernel(ps, dst_p, ev_p, src2)
  ids, vals = _sort_kernel(part, b16)
  xf = x.reshape(N * D)
  outf = _gather_kernel(xf, ids, vals)
  return outf[:K * D].reshape(K, D)
